# Initial kernel scaffold; baseline (speedup 1.0000x reference)
import functools

import jax
import jax.numpy as jnp
from jax import lax
from jax.experimental import pallas as pl
from jax.experimental.pallas import tpu as pltpu
from jax.experimental.pallas import tpu_sc as plsc

B = 4096
V = 100000
D = 16
NF = 26
ND = 13
L = 50

NC = 2
NSUB = 16
NW = NC * NSUB
RPW = B // NW

SP_N = RPW * NF
SQ_N = RPW * L
SP_CH = SP_N // 128
SQ_CH = SQ_N // 128
SQ_HALF = SQ_CH // 2

_sc_mesh = plsc.VectorSubcoreMesh(
    core_axis_name="c", subcore_axis_name="s", num_cores=NC, num_subcores=NSUB)


@functools.partial(
    pl.kernel,
    out_type=(
        jax.ShapeDtypeStruct((B * NF // 128, 128, D), jnp.float32),
        jax.ShapeDtypeStruct((B * L // 128, 128, D), jnp.float32),
    ),
    mesh=_sc_mesh,
    scratch_types=[
        pltpu.VMEM((SP_N,), jnp.int32),
        pltpu.VMEM((SQ_N,), jnp.int32),
        pltpu.VMEM((SP_N,), jnp.int32),
        pltpu.VMEM((SP_CH, 128), jnp.int32),
        pltpu.VMEM((SQ_CH, 128), jnp.int32),
        pltpu.VMEM((SP_CH, 128, D), jnp.float32),
        pltpu.SemaphoreType.DMA,
    ],
)
def _sc_gather(sp_ids, sq_ids, offs, table, out_sp, out_sq,
               spv, sqv, offv, idx_sp, idx_sq, rows, sem):
    wid = lax.axis_index("s") * NC + lax.axis_index("c")
    sp_base = wid * SP_N
    sq_base = wid * SQ_N

    pltpu.sync_copy(sp_ids.at[pl.ds(sp_base, SP_N)], spv)
    pltpu.sync_copy(sq_ids.at[pl.ds(sq_base, SQ_N)], sqv)
    pltpu.sync_copy(offs, offv)

    def prep_sp(c, carry):
        for j in range(8):
            p = c * 128 + j * 16
            idx_sp[c, pl.ds(j * 16, 16)] = (
                spv[pl.ds(p, 16)] + offv[pl.ds(p, 16)])
        return carry
    lax.fori_loop(0, SP_CH, prep_sp, 0)

    seq_off = jnp.int32(NF * V)

    def prep_sq(c, carry):
        for j in range(8):
            p = c * 128 + j * 16
            idx_sq[c, pl.ds(j * 16, 16)] = sqv[pl.ds(p, 16)] + seq_off
        return carry
    lax.fori_loop(0, SQ_CH, prep_sq, 0)

    pltpu.async_copy(table.at[idx_sp], rows, sem).wait()
    pltpu.sync_copy(rows, out_sp.at[pl.ds(wid * SP_CH, SP_CH)])

    for h in range(2):
        pltpu.async_copy(
            table.at[idx_sq.at[pl.ds(h * SQ_HALF, SQ_HALF)]],
            rows.at[pl.ds(0, SQ_HALF)], sem).wait()
        pltpu.sync_copy(
            rows.at[pl.ds(0, SQ_HALF)],
            out_sq.at[pl.ds(wid * SQ_CH + h * SQ_HALF, SQ_HALF)])


def _tc_body(sf, sq8, dense, seqf, msel, row0,
             w1a, w1b, w1c, b1, w2, b2, wdt, bd, wwt, bw, out):
    x_sf = sf[...]
    x_sq = sq8[...]
    d = dense[...]
    s = seqf[...]

    n0 = jnp.sum((s == 0.0).astype(jnp.float32), axis=1, keepdims=True)
    pooled_sum = jax.lax.dot_general(
        x_sq, msel[...], (((1,), (0,)), ((), ())),
        precision=jax.lax.Precision.HIGHEST,
        preferred_element_type=jnp.float32) - n0 * row0[...]
    length = jnp.float32(L) - n0
    pooled = pooled_sum / (length + 1e-8)

    def mm(a, b):
        return jax.lax.dot_general(
            a, b, (((1,), (0,)), ((), ())),
            precision=jax.lax.Precision.HIGHEST,
            preferred_element_type=jnp.float32)

    h1 = mm(x_sf, w1a[...]) + mm(pooled, w1b[...]) + mm(d, w1c[...]) + b1[...]
    h1 = jnp.maximum(h1, 0.0)
    h2 = jnp.maximum(mm(h1, w2[...]) + b2[...], 0.0)
    deep = jnp.sum(h2 * wdt[...], axis=1, keepdims=True) + bd[...]
    wide = jnp.sum(d * wwt[...], axis=1, keepdims=True) + bw[...]
    z = deep + wide
    out[...] = 1.0 / (1.0 + jnp.exp(-z))


def _tc_call(sf, sq8, dense, seqf, msel, row0,
             w1a, w1b, w1c, b1, w2, b2, wdt, bd, wwt, bw):
    BM = 512
    grid = (B // BM,)

    def row_spec(cols):
        return pl.BlockSpec((BM, cols), lambda i: (i, 0))

    def full_spec(r, c):
        return pl.BlockSpec((r, c), lambda i: (0, 0))

    return pl.pallas_call(
        _tc_body,
        grid=grid,
        in_specs=[
            row_spec(NF * D),
            row_spec(L * D),
            row_spec(ND),
            row_spec(L),
            full_spec(L * D, D),
            full_spec(1, D),
            full_spec(NF * D, 256),
            full_spec(D, 256),
            full_spec(ND, 256),
            full_spec(1, 256),
            full_spec(256, 128),
            full_spec(1, 128),
            full_spec(1, 128),
            full_spec(1, 1),
            full_spec(1, ND),
            full_spec(1, 1),
        ],
        out_specs=pl.BlockSpec((BM, 1), lambda i: (i, 0)),
        out_shape=jax.ShapeDtypeStruct((B, 1), jnp.float32),
    )(sf, sq8, dense, seqf, msel, row0,
      w1a, w1b, w1c, b1, w2, b2, wdt, bd, wwt, bw)


def kernel(inputs, emb_tables, W1, b1, W2, b2, Wd, bd, Ww, bw):
    sp_ids = inputs[:, :NF].astype(jnp.int32).reshape(-1)
    dense = inputs[:, NF:NF + ND]
    seqf = inputs[:, NF + ND:]
    sq_ids = seqf.astype(jnp.int32).reshape(-1)
    table = emb_tables.reshape((NF + 1) * V, D)
    offs = jnp.tile(jnp.arange(NF, dtype=jnp.int32) * V, RPW)

    out_sp, out_sq = _sc_gather(sp_ids, sq_ids, offs, table)
    sf = out_sp.reshape(B, NF * D)
    sq8 = out_sq.reshape(B, L * D)

    msel = jnp.tile(jnp.eye(D, dtype=jnp.float32), (L, 1))
    row0 = emb_tables[NF, 0:1, :]

    return _tc_call(
        sf, sq8, dense, seqf, msel, row0,
        W1[:NF * D], W1[NF * D:NF * D + D], W1[NF * D + D:],
        b1.reshape(1, 256), W2, b2.reshape(1, 128),
        Wd.reshape(1, 128), bd.reshape(1, 1),
        Ww.reshape(1, ND), bw.reshape(1, 1))


# speedup vs baseline: 2.1786x; 2.1786x over previous
import functools

import jax
import jax.numpy as jnp
from jax import lax
from jax.experimental import pallas as pl
from jax.experimental.pallas import tpu as pltpu
from jax.experimental.pallas import tpu_sc as plsc

B = 4096
V = 100000
D = 16
NF = 26
ND = 13
L = 50

NC = 2
NSUB = 16
NW = NC * NSUB
RPW = B // NW

SP_N = RPW * NF
SQ_N = RPW * L
SP_CH = SP_N // 128
SQ_CH = SQ_N // 128
SQ_HALF = SQ_CH // 2

def _sc_gather_body(sp_ids, sq_ids, offs, table, out_sp, out_sq,
                    spv, sqv, offv, idx_sp, idx_sq, rows, sem):
    wid = lax.axis_index("s") * NC + lax.axis_index("c")
    sp_base = wid * SP_N
    sq_base = wid * SQ_N

    pltpu.sync_copy(sp_ids.at[pl.ds(sp_base, SP_N)], spv)
    pltpu.sync_copy(sq_ids.at[pl.ds(sq_base, SQ_N)], sqv)
    pltpu.sync_copy(offs, offv)

    def prep_sp(c, carry):
        for j in range(8):
            p = c * 128 + j * 16
            idx_sp[c, pl.ds(j * 16, 16)] = (
                spv[pl.ds(p, 16)] + offv[pl.ds(p, 16)])
        return carry
    lax.fori_loop(0, SP_CH, prep_sp, 0)

    seq_off = jnp.int32(NF * V)

    def prep_sq(c, carry):
        for j in range(8):
            p = c * 128 + j * 16
            idx_sq[c, pl.ds(j * 16, 16)] = sqv[pl.ds(p, 16)] + seq_off
        return carry
    lax.fori_loop(0, SQ_CH, prep_sq, 0)

    copies = [pltpu.async_copy(table.at[idx_sp.at[c]], rows.at[c], sem)
              for c in range(SP_CH)]
    for cp in copies:
        cp.wait()
    pltpu.sync_copy(rows, out_sp.at[pl.ds(wid * SP_CH, SP_CH)])

    for h in range(2):
        copies = [pltpu.async_copy(
            table.at[idx_sq.at[h * SQ_HALF + c]], rows.at[c], sem)
            for c in range(SQ_HALF)]
        for cp in copies:
            cp.wait()
        pltpu.sync_copy(
            rows.at[pl.ds(0, SQ_HALF)],
            out_sq.at[pl.ds(wid * SQ_CH + h * SQ_HALF, SQ_HALF)])


@functools.lru_cache(maxsize=1)
def _sc_gather():
    mesh = plsc.VectorSubcoreMesh(
        core_axis_name="c", subcore_axis_name="s",
        num_cores=NC, num_subcores=NSUB)
    return pl.kernel(
        _sc_gather_body,
        out_type=(
            jax.ShapeDtypeStruct((B * NF // 128, 128, D), jnp.float32),
            jax.ShapeDtypeStruct((B * L // 128, 128, D), jnp.float32),
        ),
        mesh=mesh,
        compiler_params=pltpu.CompilerParams(use_tc_tiling_on_sc=False),
        scratch_types=[
            pltpu.VMEM((SP_N,), jnp.int32),
            pltpu.VMEM((SQ_N,), jnp.int32),
            pltpu.VMEM((SP_N,), jnp.int32),
            pltpu.VMEM((SP_CH, 128), jnp.int32),
            pltpu.VMEM((SQ_CH, 128), jnp.int32),
            pltpu.VMEM((SP_CH, 128, D), jnp.float32),
            pltpu.SemaphoreType.DMA,
        ],
    )


def _tc_body(sf, sq8, dense, seqf, msel, row0,
             w1a, w1b, w1c, b1, w2, b2, wdt, bd, wwt, bw, out):
    x_sf = sf[...]
    x_sq = sq8[...]
    d = dense[...]
    s = seqf[...]

    n0 = jnp.sum((s == 0.0).astype(jnp.float32), axis=1, keepdims=True)
    pooled_sum = jax.lax.dot_general(
        x_sq, msel[...], (((1,), (0,)), ((), ())),
        precision=jax.lax.Precision.HIGHEST,
        preferred_element_type=jnp.float32) - n0 * row0[...]
    length = jnp.float32(L) - n0
    pooled = pooled_sum / (length + 1e-8)

    def mm(a, b):
        return jax.lax.dot_general(
            a, b, (((1,), (0,)), ((), ())),
            precision=jax.lax.Precision.HIGHEST,
            preferred_element_type=jnp.float32)

    h1 = mm(x_sf, w1a[...]) + mm(pooled, w1b[...]) + mm(d, w1c[...]) + b1[...]
    h1 = jnp.maximum(h1, 0.0)
    h2 = jnp.maximum(mm(h1, w2[...]) + b2[...], 0.0)
    deep = jnp.sum(h2 * wdt[...], axis=1, keepdims=True) + bd[...]
    wide = jnp.sum(d * wwt[...], axis=1, keepdims=True) + bw[...]
    z = deep + wide
    out[...] = 1.0 / (1.0 + jnp.exp(-z))


def _tc_call(sf, sq8, dense, seqf, msel, row0,
             w1a, w1b, w1c, b1, w2, b2, wdt, bd, wwt, bw):
    BM = 512
    grid = (B // BM,)

    def row_spec(cols):
        return pl.BlockSpec((BM, cols), lambda i: (i, 0))

    def full_spec(r, c):
        return pl.BlockSpec((r, c), lambda i: (0, 0))

    return pl.pallas_call(
        _tc_body,
        grid=grid,
        in_specs=[
            row_spec(NF * D),
            row_spec(L * D),
            row_spec(ND),
            row_spec(L),
            full_spec(L * D, D),
            full_spec(1, D),
            full_spec(NF * D, 256),
            full_spec(D, 256),
            full_spec(ND, 256),
            full_spec(1, 256),
            full_spec(256, 128),
            full_spec(1, 128),
            full_spec(1, 128),
            full_spec(1, 1),
            full_spec(1, ND),
            full_spec(1, 1),
        ],
        out_specs=pl.BlockSpec((BM, 1), lambda i: (i, 0)),
        out_shape=jax.ShapeDtypeStruct((B, 1), jnp.float32),
    )(sf, sq8, dense, seqf, msel, row0,
      w1a, w1b, w1c, b1, w2, b2, wdt, bd, wwt, bw)


def kernel(inputs, emb_tables, W1, b1, W2, b2, Wd, bd, Ww, bw):
    sp_ids = inputs[:, :NF].astype(jnp.int32).reshape(-1)
    dense = inputs[:, NF:NF + ND]
    seqf = inputs[:, NF + ND:]
    sq_ids = seqf.astype(jnp.int32).reshape(-1)
    table = emb_tables.reshape((NF + 1) * V, D)
    offs = jnp.tile(jnp.arange(NF, dtype=jnp.int32) * V, RPW)

    out_sp, out_sq = _sc_gather()(sp_ids, sq_ids, offs, table)
    sf = out_sp.reshape(B, NF * D)
    sq8 = out_sq.reshape(B, L * D)

    msel = jnp.tile(jnp.eye(D, dtype=jnp.float32), (L, 1))
    row0 = emb_tables[NF, 0:1, :]

    return _tc_call(
        sf, sq8, dense, seqf, msel, row0,
        W1[:NF * D], W1[NF * D:NF * D + D], W1[NF * D + D:],
        b1.reshape(1, 256), W2, b2.reshape(1, 128),
        Wd.reshape(1, 128), bd.reshape(1, 1),
        Ww.reshape(1, ND), bw.reshape(1, 1))


# speedup vs baseline: 2.2465x; 1.0311x over previous
import functools

import jax
import jax.numpy as jnp
from jax import lax
from jax.experimental import pallas as pl
from jax.experimental.pallas import tpu as pltpu
from jax.experimental.pallas import tpu_sc as plsc

B = 4096
V = 100000
D = 16
NF = 26
ND = 13
L = 50

NC = 2
NSUB = 16
NW = NC * NSUB
RPW = B // NW

SP_N = RPW * NF
SQ_N = RPW * L
SP_CH = SP_N // 128
SQ_CH = SQ_N // 128
SQ_HALF = SQ_CH // 2

def _sc_gather_body(sp_ids, sq_ids, offs, table, out_sp, out_sq,
                    spv, sqv, offv, idx_sp, idx_sq, rows, sem):
    wid = lax.axis_index("s") * NC + lax.axis_index("c")
    sp_base = wid * SP_N
    sq_base = wid * SQ_N

    pltpu.sync_copy(sp_ids.at[pl.ds(sp_base, SP_N)], spv)
    pltpu.sync_copy(sq_ids.at[pl.ds(sq_base, SQ_N)], sqv)
    pltpu.sync_copy(offs, offv)

    def prep_sp(c, carry):
        for j in range(8):
            p = c * 128 + j * 16
            idx_sp[c, pl.ds(j * 16, 16)] = (
                spv[pl.ds(p, 16)] + offv[pl.ds(p, 16)])
        return carry
    lax.fori_loop(0, SP_CH, prep_sp, 0)

    seq_off = jnp.int32(NF * V)

    def prep_sq(c, carry):
        for j in range(8):
            p = c * 128 + j * 16
            idx_sq[c, pl.ds(j * 16, 16)] = sqv[pl.ds(p, 16)] + seq_off
        return carry
    lax.fori_loop(0, SQ_CH, prep_sq, 0)

    copies = [pltpu.async_copy(table.at[idx_sp.at[c]], rows.at[c], sem)
              for c in range(SP_CH)]
    for cp in copies:
        cp.wait()
    pltpu.sync_copy(rows, out_sp.at[pl.ds(wid * SP_CH, SP_CH)])

    for h in range(2):
        copies = [pltpu.async_copy(
            table.at[idx_sq.at[h * SQ_HALF + c]], rows.at[c], sem)
            for c in range(SQ_HALF)]
        for cp in copies:
            cp.wait()
        pltpu.sync_copy(
            rows.at[pl.ds(0, SQ_HALF)],
            out_sq.at[pl.ds(wid * SQ_CH + h * SQ_HALF, SQ_HALF)])


@functools.lru_cache(maxsize=1)
def _sc_gather():
    mesh = plsc.VectorSubcoreMesh(
        core_axis_name="c", subcore_axis_name="s",
        num_cores=NC, num_subcores=NSUB)
    return pl.kernel(
        _sc_gather_body,
        out_type=(
            jax.ShapeDtypeStruct((B * NF // 128, 128, D), jnp.float32),
            jax.ShapeDtypeStruct((B * L // 128, 128, D), jnp.float32),
        ),
        mesh=mesh,
        compiler_params=pltpu.CompilerParams(use_tc_tiling_on_sc=False),
        scratch_types=[
            pltpu.VMEM((SP_N,), jnp.int32),
            pltpu.VMEM((SQ_N,), jnp.int32),
            pltpu.VMEM((SP_N,), jnp.int32),
            pltpu.VMEM((SP_CH, 128), jnp.int32),
            pltpu.VMEM((SQ_CH, 128), jnp.int32),
            pltpu.VMEM((SP_CH, 128, D), jnp.float32),
            pltpu.SemaphoreType.DMA,
        ],
    )


def _tc_body(sf, sq8, dense, seqf, msel, row0,
             w1, b1, w2, b2, wd, bd, ww, bw, out):
    x_sf = sf[...]
    x_sq = sq8[...]
    d = dense[...]
    s = seqf[...]

    n0 = jnp.sum((s == 0.0).astype(jnp.float32), axis=1, keepdims=True)
    pooled_sum = jax.lax.dot_general(
        x_sq, msel[...], (((1,), (0,)), ((), ())),
        precision=jax.lax.Precision.HIGHEST,
        preferred_element_type=jnp.float32) - n0 * row0[...]
    length = jnp.float32(L) - n0
    pooled = pooled_sum / (length + 1e-8)

    dnn_in = jnp.concatenate([x_sf, pooled, d], axis=1)
    h1 = jnp.maximum(
        jnp.dot(dnn_in, w1[...], preferred_element_type=jnp.float32)
        + b1[...], 0.0)
    h2 = jnp.maximum(
        jnp.dot(h1, w2[...], preferred_element_type=jnp.float32)
        + b2[...], 0.0)
    deep = jnp.dot(h2, wd[...], preferred_element_type=jnp.float32) + bd[...]
    wide = jnp.dot(d, ww[...], preferred_element_type=jnp.float32) + bw[...]
    z = wide + deep
    out[...] = 1.0 / (1.0 + jnp.exp(-z))


def _tc_call(sf, sq8, dense, seqf, msel, row0,
             w1, b1, w2, b2, wd, bd, ww, bw):
    BM = 512
    grid = (B // BM,)

    def row_spec(cols):
        return pl.BlockSpec((BM, cols), lambda i: (i, 0))

    def full_spec(r, c):
        return pl.BlockSpec((r, c), lambda i: (0, 0))

    return pl.pallas_call(
        _tc_body,
        grid=grid,
        in_specs=[
            row_spec(NF * D),
            row_spec(L * D),
            row_spec(ND),
            row_spec(L),
            full_spec(L * D, D),
            full_spec(1, D),
            full_spec(NF * D + D + ND, 256),
            full_spec(1, 256),
            full_spec(256, 128),
            full_spec(1, 128),
            full_spec(128, 1),
            full_spec(1, 1),
            full_spec(ND, 1),
            full_spec(1, 1),
        ],
        out_specs=pl.BlockSpec((BM, 1), lambda i: (i, 0)),
        out_shape=jax.ShapeDtypeStruct((B, 1), jnp.float32),
    )(sf, sq8, dense, seqf, msel, row0,
      w1, b1, w2, b2, wd, bd, ww, bw)


def kernel(inputs, emb_tables, W1, b1, W2, b2, Wd, bd, Ww, bw):
    sp_ids = inputs[:, :NF].astype(jnp.int32).reshape(-1)
    dense = inputs[:, NF:NF + ND]
    seqf = inputs[:, NF + ND:]
    sq_ids = seqf.astype(jnp.int32).reshape(-1)
    table = emb_tables.reshape((NF + 1) * V, D)
    offs = jnp.tile(jnp.arange(NF, dtype=jnp.int32) * V, RPW)

    out_sp, out_sq = _sc_gather()(sp_ids, sq_ids, offs, table)
    sf = out_sp.reshape(B, NF * D)
    sq8 = out_sq.reshape(B, L * D)

    msel = jnp.tile(jnp.eye(D, dtype=jnp.float32), (L, 1))
    row0 = emb_tables[NF, 0:1, :]

    return _tc_call(
        sf, sq8, dense, seqf, msel, row0,
        W1, b1.reshape(1, 256), W2, b2.reshape(1, 128),
        Wd, bd.reshape(1, 1), Ww, bw.reshape(1, 1))


# speedup vs baseline: 6.1572x; 2.7408x over previous
import functools

import jax
import jax.numpy as jnp
from jax import lax
from jax.experimental import pallas as pl
from jax.experimental.pallas import tpu as pltpu
from jax.experimental.pallas import tpu_sc as plsc

B = 4096
V = 100000
D = 16
NF = 26
ND = 13
L = 50

NC = 2
NSUB = 16
NW = NC * NSUB

NROW = (NF + 1) * D
SP_ROWS = NF * D
SP_PER_W = SP_ROWS // NW
BH = B // 2


def _sc_body(tab, sp_idx, sq_idx, out_sp, out_sq,
             rowbuf, spbuf, sqbuf, outbuf, accbuf):
    wid = lax.axis_index("s") * NC + lax.axis_index("c")

    def sp_unit(k, carry):
        r = wid * SP_PER_W + k
        t = r // D
        pltpu.sync_copy(tab.at[r], rowbuf)
        pltpu.sync_copy(sp_idx.at[t], spbuf)

        def gather_grp(g, c):
            for j in range(4):
                p = (g * 4 + j) * 16
                outbuf[pl.ds(p, 16)] = plsc.load_gather(
                    rowbuf, [spbuf[pl.ds(p, 16)]])
            return c
        lax.fori_loop(0, B // 64, gather_grp, 0)
        pltpu.sync_copy(outbuf, out_sp.at[r])
        return carry
    lax.fori_loop(0, SP_PER_W, sp_unit, 0)

    d = wid % D
    half = wid // D
    pltpu.sync_copy(tab.at[SP_ROWS + d], rowbuf)

    def seq_chunk(bc, carry):
        col = half * BH + bc * 128
        pltpu.sync_copy(sq_idx.at[:, pl.ds(col, 128)], sqbuf)

        def seq_grp(j, c):
            p = j * 16
            acc = plsc.load_gather(rowbuf, [sqbuf[0, pl.ds(p, 16)]])
            for l in range(1, L):
                acc = acc + plsc.load_gather(rowbuf, [sqbuf[l, pl.ds(p, 16)]])
            accbuf[pl.ds(p, 16)] = acc
            return c
        lax.fori_loop(0, 8, seq_grp, 0)
        pltpu.sync_copy(accbuf, out_sq.at[d, pl.ds(col, 128)])
        return carry
    lax.fori_loop(0, BH // 128, seq_chunk, 0)


@functools.lru_cache(maxsize=1)
def _sc_gather():
    mesh = plsc.VectorSubcoreMesh(
        core_axis_name="c", subcore_axis_name="s",
        num_cores=NC, num_subcores=NSUB)
    return pl.kernel(
        _sc_body,
        out_type=(
            jax.ShapeDtypeStruct((SP_ROWS, B), jnp.float32),
            jax.ShapeDtypeStruct((D, B), jnp.float32),
        ),
        mesh=mesh,
        compiler_params=pltpu.CompilerParams(
            use_tc_tiling_on_sc=False, needs_layout_passes=False),
        scratch_types=[
            pltpu.VMEM((V,), jnp.float32),
            pltpu.VMEM((B,), jnp.int32),
            pltpu.VMEM((L, 128), jnp.int32),
            pltpu.VMEM((B,), jnp.float32),
            pltpu.VMEM((128,), jnp.float32),
        ],
    )


def _tc_body(sf, psum, dense, seqf, row0,
             w1, b1, w2, b2, wd, bd, ww, bw, out):
    x_sf = sf[...]
    d = dense[...]
    s = seqf[...]

    n0 = jnp.sum((s == 0.0).astype(jnp.float32), axis=1, keepdims=True)
    pooled_sum = psum[...] - n0 * row0[...]
    length = jnp.float32(L) - n0
    pooled = pooled_sum / (length + 1e-8)

    dnn_in = jnp.concatenate([x_sf, pooled, d], axis=1)
    h1 = jnp.maximum(
        jnp.dot(dnn_in, w1[...], preferred_element_type=jnp.float32)
        + b1[...], 0.0)
    h2 = jnp.maximum(
        jnp.dot(h1, w2[...], preferred_element_type=jnp.float32)
        + b2[...], 0.0)
    deep = jnp.dot(h2, wd[...], preferred_element_type=jnp.float32) + bd[...]
    wide = jnp.dot(d, ww[...], preferred_element_type=jnp.float32) + bw[...]
    z = wide + deep
    out[...] = 1.0 / (1.0 + jnp.exp(-z))


def _tc_call(sf, psum, dense, seqf, row0,
             w1, b1, w2, b2, wd, bd, ww, bw):
    BM = 512
    grid = (B // BM,)

    def row_spec(cols):
        return pl.BlockSpec((BM, cols), lambda i: (i, 0))

    def full_spec(r, c):
        return pl.BlockSpec((r, c), lambda i: (0, 0))

    return pl.pallas_call(
        _tc_body,
        grid=grid,
        in_specs=[
            row_spec(NF * D),
            row_spec(D),
            row_spec(ND),
            row_spec(L),
            full_spec(1, D),
            full_spec(NF * D + D + ND, 256),
            full_spec(1, 256),
            full_spec(256, 128),
            full_spec(1, 128),
            full_spec(128, 1),
            full_spec(1, 1),
            full_spec(ND, 1),
            full_spec(1, 1),
        ],
        out_specs=pl.BlockSpec((BM, 1), lambda i: (i, 0)),
        out_shape=jax.ShapeDtypeStruct((B, 1), jnp.float32),
    )(sf, psum, dense, seqf, row0,
      w1, b1, w2, b2, wd, bd, ww, bw)


def kernel(inputs, emb_tables, W1, b1, W2, b2, Wd, bd, Ww, bw):
    tab = jnp.swapaxes(emb_tables, 1, 2).reshape(NROW, V)
    sp_idx = inputs[:, :NF].astype(jnp.int32).T
    dense = inputs[:, NF:NF + ND]
    seqf = inputs[:, NF + ND:]
    sq_idx = seqf.astype(jnp.int32).T

    out_sp, out_sq = _sc_gather()(tab, sp_idx, sq_idx)
    sf = out_sp.reshape(NF, D, B).transpose(2, 0, 1).reshape(B, NF * D)
    psum = out_sq.T

    row0 = emb_tables[NF, 0:1, :]

    return _tc_call(
        sf, psum, dense, seqf, row0,
        W1, b1.reshape(1, 256), W2, b2.reshape(1, 128),
        Wd, bd.reshape(1, 1), Ww, bw.reshape(1, 1))


# speedup vs baseline: 14.9823x; 2.4333x over previous
import functools

import jax
import jax.numpy as jnp
from jax import lax
from jax.experimental import pallas as pl
from jax.experimental.pallas import tpu as pltpu
from jax.experimental.pallas import tpu_sc as plsc

B = 4096
V = 100000
D = 16
NF = 26
ND = 13
L = 50

NC = 2
NSUB = 16
NW = NC * NSUB

NROW = (NF + 1) * D
SP_ROWS = NF * D
SP_PER_W = SP_ROWS // NW
BH = B // 2


def _sc_body(tab, sp_idx, sq_idx, out_sp, out_sq,
             rowbuf, spbuf, sqbuf, outbuf, accbuf):
    wid = lax.axis_index("s") * NC + lax.axis_index("c")

    def sp_unit(k, carry):
        r = wid * SP_PER_W + k
        t = r // D
        pltpu.sync_copy(tab.at[r], rowbuf)
        pltpu.sync_copy(sp_idx.at[t], spbuf)

        def gather_grp(g, c):
            for j in range(4):
                p = (g * 4 + j) * 16
                outbuf[pl.ds(p, 16)] = plsc.load_gather(
                    rowbuf, [spbuf[pl.ds(p, 16)]])
            return c
        lax.fori_loop(0, B // 64, gather_grp, 0)
        pltpu.sync_copy(outbuf, out_sp.at[r])
        return carry
    lax.fori_loop(0, SP_PER_W, sp_unit, 0)

    d = wid % D
    half = wid // D
    pltpu.sync_copy(tab.at[SP_ROWS + d], rowbuf)

    def seq_chunk(bc, carry):
        col = half * BH + bc * 128
        pltpu.sync_copy(sq_idx.at[:, pl.ds(col, 128)], sqbuf)

        def seq_grp(j, c):
            p = j * 16
            acc = plsc.load_gather(rowbuf, [sqbuf[0, pl.ds(p, 16)]])
            for l in range(1, L):
                acc = acc + plsc.load_gather(rowbuf, [sqbuf[l, pl.ds(p, 16)]])
            accbuf[pl.ds(p, 16)] = acc
            return c
        lax.fori_loop(0, 8, seq_grp, 0)
        pltpu.sync_copy(accbuf, out_sq.at[d, pl.ds(col, 128)])
        return carry
    lax.fori_loop(0, BH // 128, seq_chunk, 0)


@functools.lru_cache(maxsize=1)
def _sc_gather():
    mesh = plsc.VectorSubcoreMesh(
        core_axis_name="c", subcore_axis_name="s",
        num_cores=NC, num_subcores=NSUB)
    return pl.kernel(
        _sc_body,
        out_type=(
            jax.ShapeDtypeStruct((SP_ROWS, B), jnp.float32),
            jax.ShapeDtypeStruct((D, B), jnp.float32),
        ),
        mesh=mesh,
        compiler_params=pltpu.CompilerParams(
            use_tc_tiling_on_sc=True, needs_layout_passes=False),
        scratch_types=[
            pltpu.VMEM((V,), jnp.float32),
            pltpu.VMEM((B,), jnp.int32),
            pltpu.VMEM((L, 128), jnp.int32),
            pltpu.VMEM((B,), jnp.float32),
            pltpu.VMEM((128,), jnp.float32),
        ],
    )


def _tc_body(sf, psum, dense, seqf, row0,
             w1, b1, w2, b2, wd, bd, ww, bw, out):
    x_sf = sf[...]
    d = dense[...]
    s = seqf[...]

    n0 = jnp.sum((s == 0.0).astype(jnp.float32), axis=1, keepdims=True)
    pooled_sum = psum[...] - n0 * row0[...]
    length = jnp.float32(L) - n0
    pooled = pooled_sum / (length + 1e-8)

    dnn_in = jnp.concatenate([x_sf, pooled, d], axis=1)
    h1 = jnp.maximum(
        jnp.dot(dnn_in, w1[...], preferred_element_type=jnp.float32)
        + b1[...], 0.0)
    h2 = jnp.maximum(
        jnp.dot(h1, w2[...], preferred_element_type=jnp.float32)
        + b2[...], 0.0)
    deep = jnp.dot(h2, wd[...], preferred_element_type=jnp.float32) + bd[...]
    wide = jnp.dot(d, ww[...], preferred_element_type=jnp.float32) + bw[...]
    z = wide + deep
    out[...] = 1.0 / (1.0 + jnp.exp(-z))


def _tc_call(sf, psum, dense, seqf, row0,
             w1, b1, w2, b2, wd, bd, ww, bw):
    BM = 512
    grid = (B // BM,)

    def row_spec(cols):
        return pl.BlockSpec((BM, cols), lambda i: (i, 0))

    def full_spec(r, c):
        return pl.BlockSpec((r, c), lambda i: (0, 0))

    return pl.pallas_call(
        _tc_body,
        grid=grid,
        in_specs=[
            row_spec(NF * D),
            row_spec(D),
            row_spec(ND),
            row_spec(L),
            full_spec(1, D),
            full_spec(NF * D + D + ND, 256),
            full_spec(1, 256),
            full_spec(256, 128),
            full_spec(1, 128),
            full_spec(128, 1),
            full_spec(1, 1),
            full_spec(ND, 1),
            full_spec(1, 1),
        ],
        out_specs=pl.BlockSpec((BM, 1), lambda i: (i, 0)),
        out_shape=jax.ShapeDtypeStruct((B, 1), jnp.float32),
    )(sf, psum, dense, seqf, row0,
      w1, b1, w2, b2, wd, bd, ww, bw)


def kernel(inputs, emb_tables, W1, b1, W2, b2, Wd, bd, Ww, bw):
    tab = jnp.swapaxes(emb_tables, 1, 2).reshape(NROW, V)
    sp_idx = inputs[:, :NF].astype(jnp.int32).T
    dense = inputs[:, NF:NF + ND]
    seqf = inputs[:, NF + ND:]
    sq_idx = seqf.astype(jnp.int32).T

    out_sp, out_sq = _sc_gather()(tab, sp_idx, sq_idx)
    sf = out_sp.reshape(NF, D, B).transpose(2, 0, 1).reshape(B, NF * D)
    psum = out_sq.T

    row0 = emb_tables[NF, 0:1, :]

    return _tc_call(
        sf, psum, dense, seqf, row0,
        W1, b1.reshape(1, 256), W2, b2.reshape(1, 128),
        Wd, bd.reshape(1, 1), Ww, bw.reshape(1, 1))


# speedup vs baseline: 15.8564x; 1.0583x over previous
import functools

import jax
import jax.numpy as jnp
from jax import lax
from jax.experimental import pallas as pl
from jax.experimental.pallas import tpu as pltpu
from jax.experimental.pallas import tpu_sc as plsc

B = 4096
V = 100000
D = 16
NF = 26
ND = 13
L = 50

NC = 2
NSUB = 16
NW = NC * NSUB

NROW = (NF + 1) * D
SP_ROWS = NF * D
SP_PER_W = SP_ROWS // NW
BH = B // 2


def _sc_body(tab, sp_idx, sq_idx, out_sp, out_sq,
             rowbuf, spbuf, sqbuf, outbuf, accbuf):
    wid = lax.axis_index("s") * NC + lax.axis_index("c")

    def sp_unit(k, carry):
        r = wid * SP_PER_W + k
        t = r // D
        pltpu.sync_copy(tab.at[r], rowbuf)
        pltpu.sync_copy(sp_idx.at[t], spbuf)

        def gather_grp(g, c):
            for j in range(4):
                p = (g * 4 + j) * 16
                outbuf[pl.ds(p, 16)] = plsc.load_gather(
                    rowbuf, [spbuf[pl.ds(p, 16)]])
            return c
        lax.fori_loop(0, B // 64, gather_grp, 0)
        pltpu.sync_copy(outbuf, out_sp.at[r])
        return carry
    lax.fori_loop(0, SP_PER_W, sp_unit, 0)

    d = wid % D
    half = wid // D
    pltpu.sync_copy(tab.at[SP_ROWS + d], rowbuf)

    def seq_chunk(bc, carry):
        col = half * BH + bc * 128
        pltpu.sync_copy(sq_idx.at[:, pl.ds(col, 128)], sqbuf)

        def seq_grp(j, c):
            p = j * 16
            acc = plsc.load_gather(rowbuf, [sqbuf[0, pl.ds(p, 16)]])
            for l in range(1, L):
                acc = acc + plsc.load_gather(rowbuf, [sqbuf[l, pl.ds(p, 16)]])
            accbuf[pl.ds(p, 16)] = acc
            return c
        lax.fori_loop(0, 8, seq_grp, 0)
        pltpu.sync_copy(accbuf, out_sq.at[d, pl.ds(col, 128)])
        return carry
    lax.fori_loop(0, BH // 128, seq_chunk, 0)


@functools.lru_cache(maxsize=1)
def _sc_gather():
    mesh = plsc.VectorSubcoreMesh(
        core_axis_name="c", subcore_axis_name="s",
        num_cores=NC, num_subcores=NSUB)
    return pl.kernel(
        _sc_body,
        out_type=(
            jax.ShapeDtypeStruct((SP_ROWS, B), jnp.float32),
            jax.ShapeDtypeStruct((D, B), jnp.float32),
        ),
        mesh=mesh,
        compiler_params=pltpu.CompilerParams(
            use_tc_tiling_on_sc=True, needs_layout_passes=False),
        scratch_types=[
            pltpu.VMEM((V,), jnp.float32),
            pltpu.VMEM((B,), jnp.int32),
            pltpu.VMEM((L, 128), jnp.int32),
            pltpu.VMEM((B,), jnp.float32),
            pltpu.VMEM((128,), jnp.float32),
        ],
    )


def _tc_body(sf, psum, dense, seqf, row0,
             w1, b1, w2, b2, wd, bd, ww, bw, out):
    x_sf = jnp.transpose(sf[...])
    d = dense[...]
    s = seqf[...]

    n0 = jnp.sum((s == 0.0).astype(jnp.float32), axis=1, keepdims=True)
    pooled_sum = jnp.transpose(psum[...]) - n0 * row0[...]
    length = jnp.float32(L) - n0
    pooled = pooled_sum / (length + 1e-8)

    dnn_in = jnp.concatenate([x_sf, pooled, d], axis=1)
    h1 = jnp.maximum(
        jnp.dot(dnn_in, w1[...], preferred_element_type=jnp.float32)
        + b1[...], 0.0)
    h2 = jnp.maximum(
        jnp.dot(h1, w2[...], preferred_element_type=jnp.float32)
        + b2[...], 0.0)
    deep = jnp.dot(h2, wd[...], preferred_element_type=jnp.float32) + bd[...]
    wide = jnp.dot(d, ww[...], preferred_element_type=jnp.float32) + bw[...]
    z = wide + deep
    out[...] = 1.0 / (1.0 + jnp.exp(-z))


def _tc_call(sf, psum, dense, seqf, row0,
             w1, b1, w2, b2, wd, bd, ww, bw):
    BM = 512
    grid = (B // BM,)

    def row_spec(cols):
        return pl.BlockSpec((BM, cols), lambda i: (i, 0))

    def full_spec(r, c):
        return pl.BlockSpec((r, c), lambda i: (0, 0))

    return pl.pallas_call(
        _tc_body,
        grid=grid,
        in_specs=[
            pl.BlockSpec((NF * D, BM), lambda i: (0, i)),
            pl.BlockSpec((D, BM), lambda i: (0, i)),
            row_spec(ND),
            row_spec(L),
            full_spec(1, D),
            full_spec(NF * D + D + ND, 256),
            full_spec(1, 256),
            full_spec(256, 128),
            full_spec(1, 128),
            full_spec(128, 1),
            full_spec(1, 1),
            full_spec(ND, 1),
            full_spec(1, 1),
        ],
        out_specs=pl.BlockSpec((BM, 1), lambda i: (i, 0)),
        out_shape=jax.ShapeDtypeStruct((B, 1), jnp.float32),
    )(sf, psum, dense, seqf, row0,
      w1, b1, w2, b2, wd, bd, ww, bw)


def kernel(inputs, emb_tables, W1, b1, W2, b2, Wd, bd, Ww, bw):
    tab = jnp.swapaxes(emb_tables, 1, 2).reshape(NROW, V)
    sp_idx = inputs[:, :NF].astype(jnp.int32).T
    dense = inputs[:, NF:NF + ND]
    seqf = inputs[:, NF + ND:]
    sq_idx = seqf.astype(jnp.int32).T

    out_sp, out_sq = _sc_gather()(tab, sp_idx, sq_idx)

    row0 = emb_tables[NF, 0:1, :]

    return _tc_call(
        out_sp, out_sq, dense, seqf, row0,
        W1, b1.reshape(1, 256), W2, b2.reshape(1, 128),
        Wd, bd.reshape(1, 1), Ww, bw.reshape(1, 1))
